# baseline (device time: 34182 ns/iter reference)
import functools

import jax
import jax.numpy as jnp
from jax import lax
from jax.experimental import pallas as pl
from jax.experimental.pallas import tpu as pltpu

N_DEV = 8
B = 2
SQ = 256
SKV_LOC = 256
HQ = 4
DH = 64
D_MODEL = 512
BLK = 64
SCALE = 0.125
NEG = -1e9

XOR_MASKS = (1, 3, 4)
N_STEPS = len(XOR_MASKS)


def kernel(x, Wq, K_ext, V_ext, Wo):
    def body(
        x_ref,
        wq_ref,
        k_ref,
        v_ref,
        wo_ref,
        out_ref,
        ctx_send,
        ctx_recv,
        l_send,
        l_recv,
        ctx_send_sems,
        ctx_recv_sems,
        l_send_sems,
        l_recv_sems,
    ):
        my = lax.axis_index("i")

        barrier = pltpu.get_barrier_semaphore()
        for m in XOR_MASKS:
            partner = jnp.bitwise_xor(my, m)
            pl.semaphore_signal(
                barrier,
                inc=1,
                device_id=(partner,),
                device_id_type=pl.DeviceIdType.MESH,
            )
        pl.semaphore_wait(barrier, N_STEPS)

        q_all = jnp.dot(
            x_ref[:].reshape(B * SQ, D_MODEL),
            wq_ref[:],
            preferred_element_type=jnp.float32,
        )

        qb = lax.broadcasted_iota(jnp.int32, (SQ, SKV_LOC), 0) // BLK
        kb = my * (SKV_LOC // BLK) + lax.broadcasted_iota(
            jnp.int32, (SQ, SKV_LOC), 1
        ) // BLK
        mask = (qb == kb) | (kb == 0) | (((qb + kb) % 3) == 0)

        ctx_rows = []
        l_rows = []
        for b in range(B):
            ctx_cols = []
            for h in range(HQ):
                q_bh = q_all[b * SQ : (b + 1) * SQ, h * DH : (h + 1) * DH]
                k_bh = k_ref[b, :, h, :]
                s = lax.dot_general(
                    q_bh,
                    k_bh,
                    (((1,), (1,)), ((), ())),
                    preferred_element_type=jnp.float32,
                )
                w = jnp.exp(jnp.where(mask, s * SCALE, NEG))
                l_rows.append(jnp.sum(w, axis=1))
                ctx_cols.append(
                    jnp.dot(w, v_ref[b, :, h, :], preferred_element_type=jnp.float32)
                )
            ctx_rows.append(jnp.concatenate(ctx_cols, axis=1))
        acc_ctx = jnp.stack(ctx_rows)
        acc_l = jnp.stack(l_rows)

        for s_idx, m in enumerate(XOR_MASKS):
            partner = jnp.bitwise_xor(my, m)
            ctx_send[s_idx] = acc_ctx
            l_send[s_idx] = acc_l
            rdma_ctx = pltpu.make_async_remote_copy(
                src_ref=ctx_send.at[s_idx],
                dst_ref=ctx_recv.at[s_idx],
                send_sem=ctx_send_sems.at[s_idx],
                recv_sem=ctx_recv_sems.at[s_idx],
                device_id=(partner,),
                device_id_type=pl.DeviceIdType.MESH,
            )
            rdma_l = pltpu.make_async_remote_copy(
                src_ref=l_send.at[s_idx],
                dst_ref=l_recv.at[s_idx],
                send_sem=l_send_sems.at[s_idx],
                recv_sem=l_recv_sems.at[s_idx],
                device_id=(partner,),
                device_id_type=pl.DeviceIdType.MESH,
            )
            rdma_ctx.start()
            rdma_l.start()
            rdma_ctx.wait()
            rdma_l.wait()
            acc_ctx = acc_ctx + ctx_recv[s_idx]
            acc_l = acc_l + l_recv[s_idx]

        recip = 1.0 / acc_l
        flat_rows = []
        for b in range(B):
            cols = []
            for h in range(HQ):
                blk = acc_ctx[b, :, h * DH : (h + 1) * DH]
                r = recip[b * HQ + h, :][:, None]
                cols.append(blk * r)
            flat_rows.append(jnp.concatenate(cols, axis=1))
        flat = jnp.concatenate(flat_rows, axis=0)
        out = jnp.dot(flat, wo_ref[:], preferred_element_type=jnp.float32)
        out_ref[:] = out.reshape(B, SQ, D_MODEL)

        @functools.partial(
            pl.run_scoped, second_barrier=pltpu.SemaphoreType.REGULAR
        )
        def _(second_barrier):
            for m in XOR_MASKS:
                partner = jnp.bitwise_xor(my, m)
                pl.semaphore_signal(
                    second_barrier,
                    inc=1,
                    device_id=(partner,),
                    device_id_type=pl.DeviceIdType.MESH,
                )
            pl.semaphore_wait(second_barrier, N_STEPS)

    return pl.pallas_call(
        body,
        out_shape=jax.ShapeDtypeStruct((B, SQ, D_MODEL), jnp.float32),
        in_specs=[pl.BlockSpec(memory_space=pltpu.VMEM)] * 5,
        out_specs=pl.BlockSpec(memory_space=pltpu.VMEM),
        scratch_shapes=[
            pltpu.VMEM((N_STEPS, B, SQ, HQ * DH), jnp.float32),
            pltpu.VMEM((N_STEPS, B, SQ, HQ * DH), jnp.float32),
            pltpu.VMEM((N_STEPS, B * HQ, SQ), jnp.float32),
            pltpu.VMEM((N_STEPS, B * HQ, SQ), jnp.float32),
            pltpu.SemaphoreType.DMA((N_STEPS,)),
            pltpu.SemaphoreType.DMA((N_STEPS,)),
            pltpu.SemaphoreType.DMA((N_STEPS,)),
            pltpu.SemaphoreType.DMA((N_STEPS,)),
        ],
        compiler_params=pltpu.CompilerParams(collective_id=0),
    )(x, Wq, K_ext, V_ext, Wo)


# device time: 25752 ns/iter; 1.3274x vs baseline; 1.3274x over previous
import functools

import jax
import jax.numpy as jnp
from jax import lax
from jax.experimental import pallas as pl
from jax.experimental.pallas import tpu as pltpu

N_DEV = 8
B = 2
SQ = 256
SKV_LOC = 256
HQ = 4
DH = 64
D_MODEL = 512
BLK = 64
SCALE = 0.125
NEG = -1e9

XOR_MASKS = (1, 3, 4)
N_STEPS = len(XOR_MASKS)


def kernel(x, Wq, K_ext, V_ext, Wo):
    def body(
        x_ref,
        wq_ref,
        k_ref,
        v_ref,
        wo_ref,
        out_ref,
        ctx_send,
        ctx_recv,
        l_send,
        l_recv,
        ctx_send_sems,
        ctx_recv_sems,
        l_send_sems,
        l_recv_sems,
    ):
        my = lax.axis_index("i")

        barrier = pltpu.get_barrier_semaphore()
        for m in XOR_MASKS:
            partner = jnp.bitwise_xor(my, m)
            pl.semaphore_signal(
                barrier,
                inc=1,
                device_id=(partner,),
                device_id_type=pl.DeviceIdType.MESH,
            )
        pl.semaphore_wait(barrier, N_STEPS)

        q_all = jnp.dot(
            x_ref[:].reshape(B * SQ, D_MODEL),
            wq_ref[:],
            preferred_element_type=jnp.float32,
        )

        qb = lax.broadcasted_iota(jnp.int32, (SQ, SKV_LOC), 0) // BLK
        kb = my * (SKV_LOC // BLK) + lax.broadcasted_iota(
            jnp.int32, (SQ, SKV_LOC), 1
        ) // BLK
        mask = (qb == kb) | (kb == 0) | (((qb + kb) % 3) == 0)

        ctx_rows = []
        l_rows = []
        for b in range(B):
            ctx_cols = []
            for h in range(HQ):
                q_bh = q_all[b * SQ : (b + 1) * SQ, h * DH : (h + 1) * DH]
                k_bh = k_ref[b, :, h, :]
                s = lax.dot_general(
                    q_bh,
                    k_bh,
                    (((1,), (1,)), ((), ())),
                    preferred_element_type=jnp.float32,
                )
                w = jnp.exp(jnp.where(mask, s * SCALE, NEG))
                l_rows.append(jnp.sum(w, axis=1))
                ctx_cols.append(
                    jnp.dot(w, v_ref[b, :, h, :], preferred_element_type=jnp.float32)
                )
            ctx_rows.append(jnp.concatenate(ctx_cols, axis=1))
        acc_ctx = jnp.stack(ctx_rows)
        acc_l = jnp.stack(l_rows)

        for s_idx, m in enumerate(XOR_MASKS):
            partner = jnp.bitwise_xor(my, m)
            ctx_send[s_idx] = acc_ctx.astype(jnp.bfloat16)
            l_send[s_idx] = acc_l
            rdma_ctx = pltpu.make_async_remote_copy(
                src_ref=ctx_send.at[s_idx],
                dst_ref=ctx_recv.at[s_idx],
                send_sem=ctx_send_sems.at[s_idx],
                recv_sem=ctx_recv_sems.at[s_idx],
                device_id=(partner,),
                device_id_type=pl.DeviceIdType.MESH,
            )
            rdma_l = pltpu.make_async_remote_copy(
                src_ref=l_send.at[s_idx],
                dst_ref=l_recv.at[s_idx],
                send_sem=l_send_sems.at[s_idx],
                recv_sem=l_recv_sems.at[s_idx],
                device_id=(partner,),
                device_id_type=pl.DeviceIdType.MESH,
            )
            rdma_ctx.start()
            rdma_l.start()
            rdma_ctx.wait()
            rdma_l.wait()
            acc_ctx = acc_ctx + ctx_recv[s_idx].astype(jnp.float32)
            acc_l = acc_l + l_recv[s_idx]

        recip = 1.0 / acc_l
        flat_rows = []
        for b in range(B):
            cols = []
            for h in range(HQ):
                blk = acc_ctx[b, :, h * DH : (h + 1) * DH]
                r = recip[b * HQ + h, :][:, None]
                cols.append(blk * r)
            flat_rows.append(jnp.concatenate(cols, axis=1))
        flat = jnp.concatenate(flat_rows, axis=0)
        out = jnp.dot(flat, wo_ref[:], preferred_element_type=jnp.float32)
        out_ref[:] = out.reshape(B, SQ, D_MODEL)

        @functools.partial(
            pl.run_scoped, second_barrier=pltpu.SemaphoreType.REGULAR
        )
        def _(second_barrier):
            for m in XOR_MASKS:
                partner = jnp.bitwise_xor(my, m)
                pl.semaphore_signal(
                    second_barrier,
                    inc=1,
                    device_id=(partner,),
                    device_id_type=pl.DeviceIdType.MESH,
                )
            pl.semaphore_wait(second_barrier, N_STEPS)

    return pl.pallas_call(
        body,
        out_shape=jax.ShapeDtypeStruct((B, SQ, D_MODEL), jnp.float32),
        in_specs=[pl.BlockSpec(memory_space=pltpu.VMEM)] * 5,
        out_specs=pl.BlockSpec(memory_space=pltpu.VMEM),
        scratch_shapes=[
            pltpu.VMEM((N_STEPS, B, SQ, HQ * DH), jnp.bfloat16),
            pltpu.VMEM((N_STEPS, B, SQ, HQ * DH), jnp.bfloat16),
            pltpu.VMEM((N_STEPS, B * HQ, SQ), jnp.float32),
            pltpu.VMEM((N_STEPS, B * HQ, SQ), jnp.float32),
            pltpu.SemaphoreType.DMA((N_STEPS,)),
            pltpu.SemaphoreType.DMA((N_STEPS,)),
            pltpu.SemaphoreType.DMA((N_STEPS,)),
            pltpu.SemaphoreType.DMA((N_STEPS,)),
        ],
        compiler_params=pltpu.CompilerParams(collective_id=0),
    )(x, Wq, K_ext, V_ext, Wo)
